# baseline (speedup 1.0000x reference)
import functools

import jax
import jax.numpy as jnp
from jax import lax
from jax.experimental import pallas as pl
from jax.experimental.pallas import tpu as pltpu
from jax.experimental.pallas import tpu_sc as plsc

_NC = 2
_NS = 16
_NW = _NC * _NS

_LANES = 16


def _gather_kernel(f_fields, d_dim, v_rows, l_len, b_batch):
    n_pairs = f_fields * d_dim
    assert n_pairs % _NW == 0
    pairs_per_w = n_pairs // _NW
    assert f_fields % _NC == 0
    f_per_sc = f_fields // _NC

    mesh = plsc.VectorSubcoreMesh(core_axis_name="c", subcore_axis_name="s")

    @functools.partial(
        pl.kernel,
        mesh=mesh,
        compiler_params=pltpu.CompilerParams(needs_layout_passes=False),
        out_type=jax.ShapeDtypeStruct((l_len, f_fields, d_dim, b_batch), jnp.float32),
        scratch_types=[
            pltpu.VMEM((v_rows,), jnp.float32),
            pltpu.VMEM((b_batch,), jnp.int32),
            pltpu.VMEM((b_batch,), jnp.int32),
            pltpu.VMEM((b_batch,), jnp.float32),
            pltpu.VMEM((b_batch,), jnp.float32),
            pltpu.SemaphoreType.DMA,
            pltpu.SemaphoreType.DMA,
            pltpu.SemaphoreType.DMA,
            pltpu.SemaphoreType.DMA,
        ],
    )
    def k(tbl_hbm, in_hbm, out_hbm, row_v, idx0, idx1, out0, out1,
          sem_i0, sem_i1, sem_o0, sem_o1):
        cid = lax.axis_index("c")
        sid = lax.axis_index("s")
        base_f = cid * f_per_sc
        idxs, outs = (idx0, idx1), (out0, out1)
        sem_i, sem_o = (sem_i0, sem_i1), (sem_o0, sem_o1)

        def do_pair(f, d):
            pltpu.sync_copy(tbl_hbm.at[f, d], row_v)
            pltpu.sync_copy(in_hbm.at[f, 0], idxs[0])
            for l in range(l_len):
                a, b = l % 2, (l + 1) % 2
                if l + 1 < l_len:
                    pltpu.make_async_copy(
                        in_hbm.at[f, l + 1], idxs[b], sem_i[b]).start()
                if l >= 1:
                    pltpu.make_async_copy(
                        in_hbm.at[f, l], idxs[a], sem_i[a]).wait()
                if l >= 2:
                    pltpu.make_async_copy(
                        outs[a], out_hbm.at[l, f, d], sem_o[a]).wait()

                idx_v, out_v = idxs[a], outs[a]

                @plsc.parallel_loop(0, b_batch, step=_LANES, unroll=16)
                def g_body(i):
                    sl = pl.ds(i, _LANES)
                    out_v[sl] = plsc.load_gather(row_v, [idx_v[sl]])

                pltpu.make_async_copy(
                    outs[a], out_hbm.at[l, f, d], sem_o[a]).start()
            pltpu.make_async_copy(
                outs[0], out_hbm.at[l_len - 2, f, d], sem_o[0]).wait()
            pltpu.make_async_copy(
                outs[1], out_hbm.at[l_len - 1, f, d], sem_o[1]).wait()

        def field_body(fi, carry):
            f = base_f + fi
            for dd in range(d_dim // _NS):
                do_pair(f, dd * _NS + sid)
            return carry

        lax.fori_loop(0, f_per_sc, field_body, 0, unroll=False)

    return k


def kernel(input, tables):
    b, l, f = input.shape
    f2, v, d = tables.shape
    tbl_t = jnp.transpose(tables, (0, 2, 1))
    in_t = jnp.transpose(input, (2, 1, 0))
    out_t = _gather_kernel(f, d, v, l, b)(tbl_t, in_t)
    return jnp.transpose(out_t, (3, 0, 1, 2))


# speedup vs baseline: 1.0305x; 1.0305x over previous
import functools

import jax
import jax.numpy as jnp
from jax import lax
from jax.experimental import pallas as pl
from jax.experimental.pallas import tpu as pltpu
from jax.experimental.pallas import tpu_sc as plsc

_NC = 2
_NS = 16
_NW = _NC * _NS

_LANES = 16


def _gather_kernel(f_fields, d_dim, v_rows, l_len, b_batch):
    n_pairs = f_fields * d_dim
    assert n_pairs % _NW == 0
    pairs_per_w = n_pairs // _NW
    assert f_fields % _NC == 0
    f_per_sc = f_fields // _NC

    mesh = plsc.VectorSubcoreMesh(core_axis_name="c", subcore_axis_name="s")

    @functools.partial(
        pl.kernel,
        mesh=mesh,
        compiler_params=pltpu.CompilerParams(needs_layout_passes=False),
        out_type=jax.ShapeDtypeStruct((l_len, f_fields, d_dim, b_batch), jnp.float32),
        scratch_types=[
            pltpu.VMEM((v_rows,), jnp.float32),
            pltpu.VMEM((b_batch,), jnp.int32),
            pltpu.VMEM((b_batch,), jnp.int32),
            pltpu.VMEM((b_batch,), jnp.float32),
            pltpu.VMEM((b_batch,), jnp.float32),
            pltpu.SemaphoreType.DMA,
            pltpu.SemaphoreType.DMA,
            pltpu.SemaphoreType.DMA,
            pltpu.SemaphoreType.DMA,
        ],
    )
    def k(tbl_hbm, in_hbm, out_hbm, row_v, idx0, idx1, out0, out1,
          sem_i0, sem_i1, sem_o0, sem_o1):
        cid = lax.axis_index("c")
        sid = lax.axis_index("s")
        wid = sid * _NC + cid
        idxs, outs = (idx0, idx1), (out0, out1)
        sem_i, sem_o = (sem_i0, sem_i1), (sem_o0, sem_o1)

        def do_pair(f, d):
            pltpu.sync_copy(tbl_hbm.at[f, d], row_v)
            pltpu.sync_copy(in_hbm.at[f, 0], idxs[0])
            for l in range(l_len):
                a, b = l % 2, (l + 1) % 2
                if l + 1 < l_len:
                    pltpu.make_async_copy(
                        in_hbm.at[f, l + 1], idxs[b], sem_i[b]).start()
                if l >= 1:
                    pltpu.make_async_copy(
                        in_hbm.at[f, l], idxs[a], sem_i[a]).wait()
                if l >= 2:
                    pltpu.make_async_copy(
                        outs[a], out_hbm.at[l, f, d], sem_o[a]).wait()

                idx_v, out_v = idxs[a], outs[a]

                @plsc.parallel_loop(0, b_batch, step=_LANES, unroll=8)
                def g_body(i):
                    sl = pl.ds(i, _LANES)
                    out_v[sl] = plsc.load_gather(row_v, [idx_v[sl]])

                pltpu.make_async_copy(
                    outs[a], out_hbm.at[l, f, d], sem_o[a]).start()
            pltpu.make_async_copy(
                outs[0], out_hbm.at[l_len - 2, f, d], sem_o[0]).wait()
            pltpu.make_async_copy(
                outs[1], out_hbm.at[l_len - 1, f, d], sem_o[1]).wait()

        def pair_body(p, carry):
            do_pair(p // d_dim, p % d_dim)
            return carry

        lax.fori_loop(wid * pairs_per_w, (wid + 1) * pairs_per_w, pair_body, 0,
                      unroll=False)

    return k


def kernel(input, tables):
    b, l, f = input.shape
    f2, v, d = tables.shape
    tbl_t = jnp.transpose(tables, (0, 2, 1))
    in_t = jnp.transpose(input, (2, 1, 0))
    out_t = _gather_kernel(f, d, v, l, b)(tbl_t, in_t)
    return jnp.transpose(out_t, (3, 0, 1, 2))
